# Initial kernel scaffold; baseline (speedup 1.0000x reference)
import jax
import jax.numpy as jnp
from jax.experimental import pallas as pl


def kernel(t, y, edge_index, edge_attr, mask):
    raise NotImplementedError("write your pallas kernel here")


# speedup vs baseline: 4.0354x; 4.0354x over previous
import jax
import jax.numpy as jnp
from jax import lax
from jax.experimental import pallas as pl
from jax.experimental.pallas import tpu as pltpu
from jax.experimental.pallas import tpu_sc as plsc

N = 100000
E = 1600000
C = 32
NC = 2
NS = 16
NW = NC * NS
NPT = N // NW
CHUNK = 4000
NCHUNK = E // CHUNK
GROUPS = CHUNK // 16
CAP = 1280
FLUSH_HI = CAP - 16
SUB = 128
EP_ROWS = 125
NEP = NPT // EP_ROWS


def _sc_body(y_hbm, e_hbm, mask_hbm, out_hbm,
             ebuf, msl, mabs, mdst, msw, rows_s, rows_d, deg, acc,
             sem_s, sem_d):
    cid = lax.axis_index("c")
    sid = lax.axis_index("s")
    wid = cid * NS + sid
    lo = wid * NPT
    hi = lo + NPT

    zf = jnp.zeros((16,), jnp.float32)
    zi = jnp.zeros((16,), jnp.int32)

    def zrow(i, _):
        acc[i, pl.ds(0, 16)] = zf
        acc[i, pl.ds(16, 16)] = zf
        return 0
    lax.fori_loop(0, NPT, zrow, 0)

    def zdeg(i, _):
        deg[pl.ds(i * 16, 16)] = zf
        return 0
    lax.fori_loop(0, deg.shape[0] // 16, zdeg, 0)

    def zidx(i, _):
        mabs[pl.ds(i * 16, 16)] = zi
        mdst[pl.ds(i * 16, 16)] = zi
        return 0
    lax.fori_loop(0, CAP // 16, zidx, 0)

    def flush(q):
        nb = (q + (SUB - 1)) // SUB
        def sub(b, _):
            off = b * SUB
            cp1 = pltpu.async_copy(y_hbm.at[mabs.at[pl.ds(off, SUB)]], rows_s, sem_s)
            cp2 = pltpu.async_copy(y_hbm.at[mdst.at[pl.ds(off, SUB)]], rows_d, sem_d)
            cp1.wait()
            cp2.wait()
            k = jnp.minimum(q - off, SUB)
            def upd(j, _):
                ix = jnp.full((16,), off + j, jnp.int32)
                sl = plsc.load_gather(msl, [ix])[0]
                wv = plsc.load_gather(msw, [ix])
                ys0 = rows_s[j, pl.ds(0, 16)]
                ys1 = rows_s[j, pl.ds(16, 16)]
                yd0 = rows_d[j, pl.ds(0, 16)]
                yd1 = rows_d[j, pl.ds(16, 16)]
                m0 = jnp.maximum(ys0 - yd0, 0.0) * wv
                m1 = jnp.maximum(ys1 - yd1, 0.0) * wv
                acc[sl, pl.ds(0, 16)] = jnp.maximum(acc[sl, pl.ds(0, 16)], m0)
                acc[sl, pl.ds(16, 16)] = jnp.maximum(acc[sl, pl.ds(16, 16)], m1)
                return 0
            lax.fori_loop(0, k, upd, 0)
            return 0
        lax.fori_loop(0, nb, sub, 0)

    def chunk_body(cix, ptr):
        pltpu.sync_copy(e_hbm.at[cix], ebuf)
        def group(g, p):
            base = g * 16
            sv = ebuf[0, pl.ds(base, 16)]
            m = (sv >= lo) & (sv < hi)
            def app(pp):
                dv = ebuf[1, pl.ds(base, 16)]
                wv = plsc.bitcast(ebuf[2, pl.ds(base, 16)], jnp.float32)
                slv = sv - lo
                plsc.addupdate_scatter(deg, [slv], wv * wv, mask=m)
                plsc.store_compressed(msl.at[pl.ds(pp, 16)], slv, mask=m)
                plsc.store_compressed(mabs.at[pl.ds(pp, 16)], sv, mask=m)
                plsc.store_compressed(mdst.at[pl.ds(pp, 16)], dv, mask=m)
                plsc.store_compressed(msw.at[pl.ds(pp, 16)], wv, mask=m)
                return pp + jnp.sum(m.astype(jnp.int32))
            p1 = lax.cond(jnp.any(m), app, lambda pp: pp, p)
            def do_flush(pp):
                flush(pp)
                return jnp.int32(0)
            p2 = lax.cond(p1 >= FLUSH_HI, do_flush, lambda pp: pp, p1)
            return p2
        return lax.fori_loop(0, GROUPS, group, ptr)

    ptr = lax.fori_loop(0, NCHUNK, chunk_body, jnp.int32(0))
    flush(ptr)

    def ep(b, _):
        pltpu.sync_copy(mask_hbm.at[wid, pl.ds(b * EP_ROWS, EP_ROWS)],
                        rows_d.at[pl.ds(0, EP_ROWS)])
        def nrow(i, _):
            n = b * EP_ROWS + i
            dv = plsc.load_gather(deg, [jnp.full((16,), n, jnp.int32)])
            inv = 1.0 / jnp.maximum(dv, 1e-30)
            a0 = acc[n, pl.ds(0, 16)]
            a1 = acc[n, pl.ds(16, 16)]
            mk0 = rows_d[i, pl.ds(0, 16)]
            mk1 = rows_d[i, pl.ds(16, 16)]
            acc[n, pl.ds(0, 16)] = (1.0 - a0 * inv) * mk0
            acc[n, pl.ds(16, 16)] = (1.0 - a1 * inv) * mk1
            return 0
        lax.fori_loop(0, EP_ROWS, nrow, 0)
        return 0
    lax.fori_loop(0, NEP, ep, 0)
    pltpu.sync_copy(acc, out_hbm.at[wid])


_launch = pl.kernel(
    _sc_body,
    out_type=jax.ShapeDtypeStruct((NW, NPT, C), jnp.float32),
    mesh=plsc.VectorSubcoreMesh(core_axis_name="c", subcore_axis_name="s",
                                num_cores=NC, num_subcores=NS),
    compiler_params=pltpu.CompilerParams(use_tc_tiling_on_sc=False,
                                         needs_layout_passes=False),
    scratch_types=[
        pltpu.VMEM((3, CHUNK), jnp.int32),
        pltpu.VMEM((CAP,), jnp.int32),
        pltpu.VMEM((CAP,), jnp.int32),
        pltpu.VMEM((CAP,), jnp.int32),
        pltpu.VMEM((CAP,), jnp.float32),
        pltpu.VMEM((SUB, C), jnp.float32),
        pltpu.VMEM((SUB, C), jnp.float32),
        pltpu.VMEM((3136,), jnp.float32),
        pltpu.VMEM((NPT, C), jnp.float32),
        pltpu.SemaphoreType.DMA,
        pltpu.SemaphoreType.DMA,
    ],
)


@jax.jit
def kernel(t, y, edge_index, edge_attr, mask):
    del t
    src = edge_index[0]
    dst = edge_index[1]
    sw = jnp.sqrt(edge_attr.astype(jnp.float32))
    swb = lax.bitcast_convert_type(sw, jnp.int32)
    packed = jnp.stack([src, dst, swb], axis=0)
    packed = packed.reshape(3, NCHUNK, CHUNK).transpose(1, 0, 2)
    y_rows = y.T
    mask_rows = mask.T.reshape(NW, NPT, C)
    out3 = _launch(y_rows, packed, mask_rows)
    return out3.reshape(N, C).T


# speedup vs baseline: 7.8434x; 1.9436x over previous
import jax
import jax.numpy as jnp
from jax import lax
from jax.experimental import pallas as pl
from jax.experimental.pallas import tpu as pltpu
from jax.experimental.pallas import tpu_sc as plsc

N = 100000
E = 1600000
C = 32
NC = 2
NS = 16
NW = NC * NS
NPT = N // NW
CHUNK = 1600
NCHUNK = E // CHUNK
GROUPS = CHUNK // 16
UN = 2
CAP = 3264
TRIG = CHUNK + 1
SUB = 32
EP_ROWS = 25
NEP = NPT // EP_ROWS


def _sc_body(y_hbm, e_hbm, mask_hbm, out_hbm,
             ebuf, msl, mabs, mdst, msw, rows_s, rows_d, deg, acc, tmpv,
             sem_e, sem_s, sem_d):
    cid = lax.axis_index("c")
    sid = lax.axis_index("s")
    wid = cid * NS + sid
    lo = wid * NPT

    zf = jnp.zeros((16,), jnp.float32)
    zi = jnp.zeros((16,), jnp.int32)
    iota0 = jnp.arange(16, dtype=jnp.int32)
    iota1 = iota0 + 16
    full15 = jnp.full((16,), 15, jnp.int32)

    def zrow(i, _):
        acc[i, pl.ds(0, 16)] = zf
        acc[i, pl.ds(16, 16)] = zf
        return 0
    lax.fori_loop(0, NPT, zrow, 0)

    def zdeg(i, _):
        deg[pl.ds(i * 16, 16)] = zf
        return 0
    lax.fori_loop(0, deg.shape[0] // 16, zdeg, 0)

    def zidx(i, _):
        msl[pl.ds(i * 16, 16)] = zi
        mabs[pl.ds(i * 16, 16)] = zi
        mdst[pl.ds(i * 16, 16)] = zi
        return 0
    lax.fori_loop(0, CAP // 16, zidx, 0)

    def issue_gather(b):
        off = b * SUB
        par = lax.rem(b, 2)
        pltpu.async_copy(y_hbm.at[mabs.at[pl.ds(off, SUB)]], rows_s.at[par], sem_s)
        pltpu.async_copy(y_hbm.at[mdst.at[pl.ds(off, SUB)]], rows_d.at[par], sem_d)

    def flush(q):
        msw[pl.ds(q, 16)] = zf
        msw[pl.ds(q + 16, 16)] = zf
        nb = (q + (SUB - 1)) // SUB
        issue_gather(0)
        def sub(b, _):
            off = b * SUB
            par = lax.rem(b, 2)
            pltpu.make_async_copy(y_hbm.at[pl.ds(0, SUB)], rows_s.at[par], sem_s).wait()
            pltpu.make_async_copy(y_hbm.at[pl.ds(0, SUB)], rows_d.at[par], sem_d).wait()
            @pl.when(b + 1 < nb)
            def _():
                issue_gather(b + 1)
            for bb in range(SUB // 16):
                for j in range(16):
                    r = bb * 16 + j
                    ix = jnp.full((16,), off + r, jnp.int32)
                    sl = plsc.load_gather(msl, [ix])
                    w = plsc.load_gather(msw, [ix])
                    ys0 = rows_s[par, r, pl.ds(0, 16)]
                    ys1 = rows_s[par, r, pl.ds(16, 16)]
                    yd0 = rows_d[par, r, pl.ds(0, 16)]
                    yd1 = rows_d[par, r, pl.ds(16, 16)]
                    m0 = jnp.maximum(ys0 - yd0, 0.0) * w
                    m1 = jnp.maximum(ys1 - yd1, 0.0) * w
                    a0 = plsc.load_gather(acc, [sl, iota0])
                    a1 = plsc.load_gather(acc, [sl, iota1])
                    plsc.store_scatter(acc, [sl, iota0], jnp.maximum(a0, m0))
                    plsc.store_scatter(acc, [sl, iota1], jnp.maximum(a1, m1))
            return 0
        lax.fori_loop(0, nb, sub, 0)

    pltpu.async_copy(e_hbm.at[0], ebuf.at[0], sem_e)

    def chunk_body(cix, ptrv):
        par = lax.rem(cix, 2)
        pltpu.make_async_copy(e_hbm.at[0], ebuf.at[par], sem_e).wait()
        @pl.when(cix + 1 < NCHUNK)
        def _():
            pltpu.async_copy(e_hbm.at[cix + 1], ebuf.at[lax.rem(cix + 1, 2)], sem_e)

        def group(gi, pv):
            for u in range(UN):
                base = (gi * UN + u) * 16
                sv = ebuf[par, 0, pl.ds(base, 16)]
                dv = ebuf[par, 1, pl.ds(base, 16)]
                wv = plsc.bitcast(ebuf[par, 2, pl.ds(base, 16)], jnp.float32)
                slv = sv - lo
                m = (slv >= 0) & (slv < NPT)
                pre = plsc.cumsum(m.astype(jnp.int32))
                dest = pv + pre - 1
                plsc.addupdate_scatter(deg, [slv], wv * wv, mask=m)
                plsc.store_scatter(msl, [dest], slv, mask=m)
                plsc.store_scatter(mabs, [dest], sv, mask=m)
                plsc.store_scatter(mdst, [dest], dv, mask=m)
                plsc.store_scatter(msw, [dest], wv, mask=m)
                tmpv[pl.ds(0, 16)] = pre
                pv = pv + plsc.load_gather(tmpv, [full15])
            return pv
        ptrv = lax.fori_loop(0, GROUPS // UN, group, ptrv)

        q = ptrv[0]
        def do_flush(_):
            flush(q)
            return zi
        return lax.cond(q >= TRIG, do_flush, lambda pv: pv, ptrv)

    ptrv = lax.fori_loop(0, NCHUNK, chunk_body, zi)
    qf = ptrv[0]
    def final_flush(_):
        flush(qf)
        return 0
    lax.cond(qf > 0, final_flush, lambda _: 0, 0)

    def ep(b, _):
        pltpu.sync_copy(mask_hbm.at[wid, pl.ds(b * EP_ROWS, EP_ROWS)],
                        rows_d.at[0].at[pl.ds(0, EP_ROWS)])
        def nrow(i, _):
            n = b * EP_ROWS + i
            dv = plsc.load_gather(deg, [jnp.full((16,), n, jnp.int32)])
            inv = 1.0 / jnp.maximum(dv, 1e-30)
            a0 = acc[n, pl.ds(0, 16)]
            a1 = acc[n, pl.ds(16, 16)]
            mk0 = rows_d[0, i, pl.ds(0, 16)]
            mk1 = rows_d[0, i, pl.ds(16, 16)]
            acc[n, pl.ds(0, 16)] = (1.0 - a0 * inv) * mk0
            acc[n, pl.ds(16, 16)] = (1.0 - a1 * inv) * mk1
            return 0
        lax.fori_loop(0, EP_ROWS, nrow, 0)
        return 0
    lax.fori_loop(0, NEP, ep, 0)
    pltpu.sync_copy(acc, out_hbm.at[wid])


_launch = pl.kernel(
    _sc_body,
    out_type=jax.ShapeDtypeStruct((NW, NPT, C), jnp.float32),
    mesh=plsc.VectorSubcoreMesh(core_axis_name="c", subcore_axis_name="s",
                                num_cores=NC, num_subcores=NS),
    compiler_params=pltpu.CompilerParams(use_tc_tiling_on_sc=False,
                                         needs_layout_passes=False),
    scratch_types=[
        pltpu.VMEM((2, 3, CHUNK), jnp.int32),
        pltpu.VMEM((CAP,), jnp.int32),
        pltpu.VMEM((CAP,), jnp.int32),
        pltpu.VMEM((CAP,), jnp.int32),
        pltpu.VMEM((CAP,), jnp.float32),
        pltpu.VMEM((2, SUB, C), jnp.float32),
        pltpu.VMEM((2, SUB, C), jnp.float32),
        pltpu.VMEM((3136,), jnp.float32),
        pltpu.VMEM((NPT, C), jnp.float32),
        pltpu.VMEM((16,), jnp.int32),
        pltpu.SemaphoreType.DMA,
        pltpu.SemaphoreType.DMA,
        pltpu.SemaphoreType.DMA,
    ],
)


@jax.jit
def kernel(t, y, edge_index, edge_attr, mask):
    del t
    src = edge_index[0]
    dst = edge_index[1]
    sw = jnp.sqrt(edge_attr.astype(jnp.float32))
    swb = lax.bitcast_convert_type(sw, jnp.int32)
    packed = jnp.stack([src, dst, swb], axis=0)
    packed = packed.reshape(3, NCHUNK, CHUNK).transpose(1, 0, 2)
    y_rows = y.T
    mask_rows = mask.T.reshape(NW, NPT, C)
    out3 = _launch(y_rows, packed, mask_rows)
    return out3.reshape(N, C).T
